# baseline (speedup 1.0000x reference)
import functools

import jax
import jax.numpy as jnp
from jax import lax
from jax.experimental import pallas as pl
from jax.experimental.pallas import tpu as pltpu
from jax.experimental.pallas import tpu_sc as plsc

N = 10000
E = 320000
F = 128

NC = 2
NS = 16
NW = NC * NS
EPW = E // NW
K = 50
S = EPW // K
RPT = N // NS

R = 400
GRID = N // R

f32 = jnp.float32
i32 = jnp.int32


_SC_PARAMS = pltpu.CompilerParams(needs_layout_passes=False,
                                  use_tc_tiling_on_sc=False)


@functools.cache
def _sc_mesh():
    return plsc.VectorSubcoreMesh(core_axis_name="c", subcore_axis_name="s",
                                  num_cores=NC, num_subcores=NS)


@functools.cache
def _get_deg_kernel():
    return pl.kernel(
        _deg_body,
        out_type=jax.ShapeDtypeStruct((NW, N), f32),
        mesh=_sc_mesh(),
        scratch_types=[
            pltpu.VMEM((EPW,), i32),
            pltpu.VMEM((N,), f32),
        ],
        compiler_params=_SC_PARAMS,
    )


def _deg_body(dst_hbm, out_hbm, dst_v, deg_v):
    w = lax.axis_index("c") * NS + lax.axis_index("s")
    pltpu.sync_copy(dst_hbm.at[pl.ds(w * EPW, EPW)], dst_v)

    zeros = jnp.zeros((16,), f32)

    def zero_body(i, carry):
        deg_v[pl.ds(i * 16, 16)] = zeros
        return carry

    lax.fori_loop(0, N // 16, zero_body, 0)

    ones = jnp.ones((16,), f32)

    def body(i, carry):
        idx = dst_v[pl.ds(i * 16, 16)]
        plsc.addupdate_scatter(deg_v, [idx], ones)
        return carry

    lax.fori_loop(0, EPW // 16, body, 0)
    pltpu.sync_copy(deg_v, out_hbm.at[w])


@functools.cache
def _get_agg_kernel():
    return pl.kernel(
        _agg_body,
        out_type=jax.ShapeDtypeStruct((NC, N, F), f32),
        mesh=_sc_mesh(),
        scratch_types=[
            pltpu.VMEM((S, K), i32),
            pltpu.VMEM((S, K), i32),
            pltpu.VMEM((K, F), f32),
            pltpu.VMEM((K, F), f32),
            pltpu.VMEM((K, F), f32),
            pltpu.VMEM((K, F), f32),
            pltpu.VMEM_SHARED((N, F), f32),
            pltpu.SemaphoreType.DMA,
            pltpu.SemaphoreType.DMA,
            pltpu.SemaphoreType.DMA,
            pltpu.SemaphoreType.DMA,
            pltpu.SemaphoreType.DMA,
            pltpu.SemaphoreType.DMA,
            pltpu.SemaphoreType.DMA,
            pltpu.SemaphoreType.DMA,
        ],
        compiler_params=_SC_PARAMS,
    )


def _agg_body(hs_hbm, srcr_hbm, dstr_hbm, zrows_hbm, out_hbm,
              src_v, dst_v, buf0, buf1, buf2, buf3, acc_sh,
              gs0, gs1, gs2, gs3, ss0, ss1, ss2, ss3):
    bufs = (buf0, buf1, buf2, buf3)
    gsems = (gs0, gs1, gs2, gs3)
    ssems = (ss0, ss1, ss2, ss3)
    c = lax.axis_index("c")
    s = lax.axis_index("s")
    w = c * NS + s
    pltpu.sync_copy(srcr_hbm.at[w], src_v)
    pltpu.sync_copy(dstr_hbm.at[w], dst_v)
    pltpu.sync_copy(zrows_hbm, acc_sh.at[pl.ds(s * RPT, RPT)])
    plsc.subcore_barrier()

    D = 4
    for b in range(D):
        pltpu.async_copy(hs_hbm.at[src_v.at[b]], bufs[b], gsems[b])

    def quad(q, carry):
        for b in range(D):
            g = D * q + b
            pltpu.make_async_copy(hs_hbm.at[src_v.at[g]], bufs[b], gsems[b]).wait()
            sc = pltpu.async_copy(bufs[b], acc_sh.at[dst_v.at[g]], ssems[b],
                                  add=True)
            sc.wait()

            @pl.when(g + D < S)
            def _():
                pltpu.async_copy(hs_hbm.at[src_v.at[g + D]], bufs[b], gsems[b])
        return carry

    lax.fori_loop(0, S // D, quad, 0)
    plsc.subcore_barrier()
    pltpu.sync_copy(acc_sh.at[pl.ds(s * RPT, RPT)],
                    out_hbm.at[c].at[pl.ds(s * RPT, RPT)])


def _tc1_body(x_ref, w1_ref, wres_ref, bres_ref, degp_ref,
              hs_ref, xres_ref, dinv_ref):
    deg = jnp.sum(degp_ref[...], axis=1, keepdims=True) + 1.0
    dinv = lax.rsqrt(deg)
    x = x_ref[...]
    h = jnp.dot(x, w1_ref[...], preferred_element_type=f32)
    hs_ref[...] = h * dinv
    xres_ref[...] = jnp.dot(x, wres_ref[...], preferred_element_type=f32) + bres_ref[...]
    dinv_ref[...] = dinv


_tc1 = pl.pallas_call(
    _tc1_body,
    grid=(GRID,),
    in_specs=[
        pl.BlockSpec((R, F), lambda i: (i, 0)),
        pl.BlockSpec((F, F), lambda i: (0, 0)),
        pl.BlockSpec((F, F), lambda i: (0, 0)),
        pl.BlockSpec((1, F), lambda i: (0, 0)),
        pl.BlockSpec((R, NW), lambda i: (i, 0)),
    ],
    out_specs=[
        pl.BlockSpec((R, F), lambda i: (i, 0)),
        pl.BlockSpec((R, F), lambda i: (i, 0)),
        pl.BlockSpec((R, 1), lambda i: (i, 0)),
    ],
    out_shape=[
        jax.ShapeDtypeStruct((N, F), f32),
        jax.ShapeDtypeStruct((N, F), f32),
        jax.ShapeDtypeStruct((N, 1), f32),
    ],
)


def _tc_mid_body(aggp_ref, hs_ref, dinv_ref, b_ref, xres_ref, w_ref, out_ref):
    dinv = dinv_ref[...]
    o = dinv * (aggp_ref[0] + aggp_ref[1] + hs_ref[...]) + b_ref[...]
    if xres_ref is not None:
        o = o + xres_ref[...]
    a = jnp.maximum(o, 0.0)
    out_ref[...] = jnp.dot(a, w_ref[...], preferred_element_type=f32) * dinv


def _make_mid(with_xres):
    def body(*refs):
        if with_xres:
            aggp, hs, dinv, b, xres, w, out = refs
        else:
            aggp, hs, dinv, b, w, out = refs
            xres = None
        _tc_mid_body(aggp, hs, dinv, b, xres, w, out)

    in_specs = [
        pl.BlockSpec((NC, R, F), lambda i: (0, i, 0)),
        pl.BlockSpec((R, F), lambda i: (i, 0)),
        pl.BlockSpec((R, 1), lambda i: (i, 0)),
        pl.BlockSpec((1, F), lambda i: (0, 0)),
    ]
    if with_xres:
        in_specs.append(pl.BlockSpec((R, F), lambda i: (i, 0)))
    in_specs.append(pl.BlockSpec((F, F), lambda i: (0, 0)))
    return pl.pallas_call(
        body,
        grid=(GRID,),
        in_specs=in_specs,
        out_specs=pl.BlockSpec((R, F), lambda i: (i, 0)),
        out_shape=jax.ShapeDtypeStruct((N, F), f32),
    )


_tc_mid_res = _make_mid(True)
_tc_mid = _make_mid(False)


def _tc_fin_body(aggp_ref, hs_ref, dinv_ref, b_ref, wlin_ref, blin_ref, out_ref):
    o = dinv_ref[...] * (aggp_ref[0] + aggp_ref[1] + hs_ref[...]) + b_ref[...]
    a = jnp.maximum(o, 0.0)
    out_ref[...] = jnp.dot(a, wlin_ref[...], preferred_element_type=f32) + blin_ref[...]


_tc_fin = pl.pallas_call(
    _tc_fin_body,
    grid=(GRID,),
    in_specs=[
        pl.BlockSpec((NC, R, F), lambda i: (0, i, 0)),
        pl.BlockSpec((R, F), lambda i: (i, 0)),
        pl.BlockSpec((R, 1), lambda i: (i, 0)),
        pl.BlockSpec((1, F), lambda i: (0, 0)),
        pl.BlockSpec((F, F), lambda i: (0, 0)),
        pl.BlockSpec((1, F), lambda i: (0, 0)),
    ],
    out_specs=pl.BlockSpec((R, F), lambda i: (i, 0)),
    out_shape=jax.ShapeDtypeStruct((N, F), f32),
)


def kernel(x, edge_index, W1, b1, W2, b2, W3, b3, Wres, bres, Wlin, blin):
    src = edge_index[0]
    dst = edge_index[1]
    src_r = src.reshape(NW, S, K)
    dst_r = dst.reshape(NW, S, K)
    zrows = jnp.zeros((RPT, F), f32)

    deg_k = _get_deg_kernel()
    agg_k = _get_agg_kernel()
    degp = deg_k(dst)
    hs1, xres, dinv = _tc1(x, W1, Wres, bres.reshape(1, F), degp.T)
    agg1 = agg_k(hs1, src_r, dst_r, zrows)
    hs2 = _tc_mid_res(agg1, hs1, dinv, b1.reshape(1, F), xres, W2)
    agg2 = agg_k(hs2, src_r, dst_r, zrows)
    hs3 = _tc_mid(agg2, hs2, dinv, b2.reshape(1, F), W3)
    agg3 = agg_k(hs3, src_r, dst_r, zrows)
    out = _tc_fin(agg3, hs3, dinv, b3.reshape(1, F), Wlin, blin.reshape(1, F))
    return out


# speedup vs baseline: 1.0767x; 1.0767x over previous
import functools

import jax
import jax.numpy as jnp
from jax import lax
from jax.experimental import pallas as pl
from jax.experimental.pallas import tpu as pltpu
from jax.experimental.pallas import tpu_sc as plsc

N = 10000
E = 320000
F = 128

NC = 2
NS = 16
NW = NC * NS
EPW = E // NW
K = 50
S = EPW // K
RPT = N // NS

R = 1000
GRID = N // R

f32 = jnp.float32
i32 = jnp.int32


_SC_PARAMS = pltpu.CompilerParams(needs_layout_passes=False,
                                  use_tc_tiling_on_sc=False)


@functools.cache
def _sc_mesh():
    return plsc.VectorSubcoreMesh(core_axis_name="c", subcore_axis_name="s",
                                  num_cores=NC, num_subcores=NS)


@functools.cache
def _get_deg_kernel():
    return pl.kernel(
        _deg_body,
        out_type=jax.ShapeDtypeStruct((NW, N), f32),
        mesh=_sc_mesh(),
        scratch_types=[
            pltpu.VMEM((EPW,), i32),
            pltpu.VMEM((N,), f32),
        ],
        compiler_params=_SC_PARAMS,
    )


def _deg_body(dst_hbm, out_hbm, dst_v, deg_v):
    w = lax.axis_index("c") * NS + lax.axis_index("s")
    pltpu.sync_copy(dst_hbm.at[pl.ds(w * EPW, EPW)], dst_v)

    zeros = jnp.zeros((16,), f32)

    def zero_body(i, carry):
        deg_v[pl.ds(i * 16, 16)] = zeros
        return carry

    lax.fori_loop(0, N // 16, zero_body, 0)

    ones = jnp.ones((16,), f32)

    def body(i, carry):
        idx = dst_v[pl.ds(i * 16, 16)]
        plsc.addupdate_scatter(deg_v, [idx], ones)
        return carry

    lax.fori_loop(0, EPW // 16, body, 0)
    pltpu.sync_copy(deg_v, out_hbm.at[w])


@functools.cache
def _get_agg_kernel():
    return pl.kernel(
        _agg_body,
        out_type=jax.ShapeDtypeStruct((NC, N, F), f32),
        mesh=_sc_mesh(),
        scratch_types=[
            pltpu.VMEM((S, K), i32),
            pltpu.VMEM((S, K), i32),
            pltpu.VMEM((K, F), f32),
            pltpu.VMEM((K, F), f32),
            pltpu.VMEM((K, F), f32),
            pltpu.VMEM((K, F), f32),
            pltpu.VMEM_SHARED((N, F), f32),
            pltpu.SemaphoreType.DMA,
            pltpu.SemaphoreType.DMA,
            pltpu.SemaphoreType.DMA,
            pltpu.SemaphoreType.DMA,
            pltpu.SemaphoreType.DMA,
            pltpu.SemaphoreType.DMA,
            pltpu.SemaphoreType.DMA,
            pltpu.SemaphoreType.DMA,
        ],
        compiler_params=_SC_PARAMS,
    )


def _agg_body(hs_hbm, srcr_hbm, dstr_hbm, zrows_hbm, out_hbm,
              src_v, dst_v, buf0, buf1, buf2, buf3, acc_sh,
              gs0, gs1, gs2, gs3, ss0, ss1, ss2, ss3):
    bufs = (buf0, buf1, buf2, buf3)
    gsems = (gs0, gs1, gs2, gs3)
    ssems = (ss0, ss1, ss2, ss3)
    c = lax.axis_index("c")
    s = lax.axis_index("s")
    w = c * NS + s
    pltpu.sync_copy(srcr_hbm.at[w], src_v)
    pltpu.sync_copy(dstr_hbm.at[w], dst_v)
    pltpu.sync_copy(zrows_hbm, acc_sh.at[pl.ds(s * RPT, RPT)])
    plsc.subcore_barrier()

    D = 4
    for b in range(D):
        pltpu.async_copy(hs_hbm.at[src_v.at[b]], bufs[b], gsems[b])

    def quad(q, carry):
        for b in range(D):
            g = D * q + b
            pltpu.make_async_copy(hs_hbm.at[src_v.at[g]], bufs[b], gsems[b]).wait()
            sc = pltpu.async_copy(bufs[b], acc_sh.at[dst_v.at[g]], ssems[b],
                                  add=True)
            sc.wait()

            @pl.when(g + D < S)
            def _():
                pltpu.async_copy(hs_hbm.at[src_v.at[g + D]], bufs[b], gsems[b])
        return carry

    lax.fori_loop(0, S // D, quad, 0)
    plsc.subcore_barrier()
    pltpu.sync_copy(acc_sh.at[pl.ds(s * RPT, RPT)],
                    out_hbm.at[c].at[pl.ds(s * RPT, RPT)])


def _tc1_body(x_ref, w1_ref, wres_ref, bres_ref, degp_ref,
              hs_ref, xres_ref, dinv_ref):
    deg = jnp.sum(degp_ref[...], axis=1, keepdims=True) + 1.0
    dinv = lax.rsqrt(deg)
    x = x_ref[...]
    h = jnp.dot(x, w1_ref[...], preferred_element_type=f32)
    hs_ref[...] = h * dinv
    xres_ref[...] = jnp.dot(x, wres_ref[...], preferred_element_type=f32) + bres_ref[...]
    dinv_ref[...] = dinv


_tc1 = pl.pallas_call(
    _tc1_body,
    grid=(GRID,),
    in_specs=[
        pl.BlockSpec((R, F), lambda i: (i, 0)),
        pl.BlockSpec((F, F), lambda i: (0, 0)),
        pl.BlockSpec((F, F), lambda i: (0, 0)),
        pl.BlockSpec((1, F), lambda i: (0, 0)),
        pl.BlockSpec((R, NW), lambda i: (i, 0)),
    ],
    out_specs=[
        pl.BlockSpec((R, F), lambda i: (i, 0)),
        pl.BlockSpec((R, F), lambda i: (i, 0)),
        pl.BlockSpec((R, 1), lambda i: (i, 0)),
    ],
    out_shape=[
        jax.ShapeDtypeStruct((N, F), f32),
        jax.ShapeDtypeStruct((N, F), f32),
        jax.ShapeDtypeStruct((N, 1), f32),
    ],
)


def _tc_mid_body(aggp_ref, hs_ref, dinv_ref, b_ref, xres_ref, w_ref, out_ref):
    dinv = dinv_ref[...]
    o = dinv * (aggp_ref[0] + aggp_ref[1] + hs_ref[...]) + b_ref[...]
    if xres_ref is not None:
        o = o + xres_ref[...]
    a = jnp.maximum(o, 0.0)
    out_ref[...] = jnp.dot(a, w_ref[...], preferred_element_type=f32) * dinv


def _make_mid(with_xres):
    def body(*refs):
        if with_xres:
            aggp, hs, dinv, b, xres, w, out = refs
        else:
            aggp, hs, dinv, b, w, out = refs
            xres = None
        _tc_mid_body(aggp, hs, dinv, b, xres, w, out)

    in_specs = [
        pl.BlockSpec((NC, R, F), lambda i: (0, i, 0)),
        pl.BlockSpec((R, F), lambda i: (i, 0)),
        pl.BlockSpec((R, 1), lambda i: (i, 0)),
        pl.BlockSpec((1, F), lambda i: (0, 0)),
    ]
    if with_xres:
        in_specs.append(pl.BlockSpec((R, F), lambda i: (i, 0)))
    in_specs.append(pl.BlockSpec((F, F), lambda i: (0, 0)))
    return pl.pallas_call(
        body,
        grid=(GRID,),
        in_specs=in_specs,
        out_specs=pl.BlockSpec((R, F), lambda i: (i, 0)),
        out_shape=jax.ShapeDtypeStruct((N, F), f32),
    )


_tc_mid_res = _make_mid(True)
_tc_mid = _make_mid(False)


def _tc_fin_body(aggp_ref, hs_ref, dinv_ref, b_ref, wlin_ref, blin_ref, out_ref):
    o = dinv_ref[...] * (aggp_ref[0] + aggp_ref[1] + hs_ref[...]) + b_ref[...]
    a = jnp.maximum(o, 0.0)
    out_ref[...] = jnp.dot(a, wlin_ref[...], preferred_element_type=f32) + blin_ref[...]


_tc_fin = pl.pallas_call(
    _tc_fin_body,
    grid=(GRID,),
    in_specs=[
        pl.BlockSpec((NC, R, F), lambda i: (0, i, 0)),
        pl.BlockSpec((R, F), lambda i: (i, 0)),
        pl.BlockSpec((R, 1), lambda i: (i, 0)),
        pl.BlockSpec((1, F), lambda i: (0, 0)),
        pl.BlockSpec((F, F), lambda i: (0, 0)),
        pl.BlockSpec((1, F), lambda i: (0, 0)),
    ],
    out_specs=pl.BlockSpec((R, F), lambda i: (i, 0)),
    out_shape=jax.ShapeDtypeStruct((N, F), f32),
)


def kernel(x, edge_index, W1, b1, W2, b2, W3, b3, Wres, bres, Wlin, blin):
    src = edge_index[0]
    dst = edge_index[1]
    src_r = src.reshape(NW, S, K)
    dst_r = dst.reshape(NW, S, K)
    zrows = jnp.zeros((RPT, F), f32)

    deg_k = _get_deg_kernel()
    agg_k = _get_agg_kernel()
    degp = deg_k(dst)
    hs1, xres, dinv = _tc1(x, W1, Wres, bres.reshape(1, F), degp.T)
    agg1 = agg_k(hs1, src_r, dst_r, zrows)
    hs2 = _tc_mid_res(agg1, hs1, dinv, b1.reshape(1, F), xres, W2)
    agg2 = agg_k(hs2, src_r, dst_r, zrows)
    hs3 = _tc_mid(agg2, hs2, dinv, b2.reshape(1, F), W3)
    agg3 = agg_k(hs3, src_r, dst_r, zrows)
    out = _tc_fin(agg3, hs3, dinv, b3.reshape(1, F), Wlin, blin.reshape(1, F))
    return out


# speedup vs baseline: 1.1057x; 1.0269x over previous
import functools

import jax
import jax.numpy as jnp
from jax import lax
from jax.experimental import pallas as pl
from jax.experimental.pallas import tpu as pltpu
from jax.experimental.pallas import tpu_sc as plsc

N = 10000
E = 320000
F = 128

NC = 2
NS = 16
NW = NC * NS
EPW = E // NW
K = 50
S = EPW // K
RPT = N // NS

R = 2000
GRID = N // R

f32 = jnp.float32
i32 = jnp.int32


_SC_PARAMS = pltpu.CompilerParams(needs_layout_passes=False,
                                  use_tc_tiling_on_sc=False)


@functools.cache
def _sc_mesh():
    return plsc.VectorSubcoreMesh(core_axis_name="c", subcore_axis_name="s",
                                  num_cores=NC, num_subcores=NS)


@functools.cache
def _get_deg_kernel():
    return pl.kernel(
        _deg_body,
        out_type=jax.ShapeDtypeStruct((NW, N), f32),
        mesh=_sc_mesh(),
        scratch_types=[
            pltpu.VMEM((EPW,), i32),
            pltpu.VMEM((N,), f32),
        ],
        compiler_params=_SC_PARAMS,
    )


def _deg_body(dst_hbm, out_hbm, dst_v, deg_v):
    w = lax.axis_index("c") * NS + lax.axis_index("s")
    pltpu.sync_copy(dst_hbm.at[pl.ds(w * EPW, EPW)], dst_v)

    zeros = jnp.zeros((16,), f32)

    def zero_body(i, carry):
        deg_v[pl.ds(i * 16, 16)] = zeros
        return carry

    lax.fori_loop(0, N // 16, zero_body, 0)

    ones = jnp.ones((16,), f32)

    def body(i, carry):
        idx = dst_v[pl.ds(i * 16, 16)]
        plsc.addupdate_scatter(deg_v, [idx], ones)
        return carry

    lax.fori_loop(0, EPW // 16, body, 0)
    pltpu.sync_copy(deg_v, out_hbm.at[w])


@functools.cache
def _get_agg_kernel():
    return pl.kernel(
        _agg_body,
        out_type=jax.ShapeDtypeStruct((NC, N, F), f32),
        mesh=_sc_mesh(),
        scratch_types=[
            pltpu.VMEM((S, K), i32),
            pltpu.VMEM((S, K), i32),
            pltpu.VMEM((K, F), f32),
            pltpu.VMEM((K, F), f32),
            pltpu.VMEM((K, F), f32),
            pltpu.VMEM((K, F), f32),
            pltpu.VMEM_SHARED((N, F), f32),
            pltpu.SemaphoreType.DMA,
            pltpu.SemaphoreType.DMA,
            pltpu.SemaphoreType.DMA,
            pltpu.SemaphoreType.DMA,
            pltpu.SemaphoreType.DMA,
            pltpu.SemaphoreType.DMA,
            pltpu.SemaphoreType.DMA,
            pltpu.SemaphoreType.DMA,
        ],
        compiler_params=_SC_PARAMS,
    )


def _agg_body(hs_hbm, srcr_hbm, dstr_hbm, zrows_hbm, out_hbm,
              src_v, dst_v, buf0, buf1, buf2, buf3, acc_sh,
              gs0, gs1, gs2, gs3, ss0, ss1, ss2, ss3):
    bufs = (buf0, buf1, buf2, buf3)
    gsems = (gs0, gs1, gs2, gs3)
    ssems = (ss0, ss1, ss2, ss3)
    c = lax.axis_index("c")
    s = lax.axis_index("s")
    w = c * NS + s
    pltpu.sync_copy(srcr_hbm.at[w], src_v)
    pltpu.sync_copy(dstr_hbm.at[w], dst_v)
    pltpu.sync_copy(zrows_hbm, acc_sh.at[pl.ds(s * RPT, RPT)])
    plsc.subcore_barrier()

    D = 4
    for b in range(D):
        pltpu.async_copy(hs_hbm.at[src_v.at[b]], bufs[b], gsems[b])

    def quad(q, carry):
        for b in range(D):
            g = D * q + b
            pltpu.make_async_copy(hs_hbm.at[src_v.at[g]], bufs[b], gsems[b]).wait()
            sc = pltpu.async_copy(bufs[b], acc_sh.at[dst_v.at[g]], ssems[b],
                                  add=True)
            sc.wait()

            @pl.when(g + D < S)
            def _():
                pltpu.async_copy(hs_hbm.at[src_v.at[g + D]], bufs[b], gsems[b])
        return carry

    lax.fori_loop(0, S // D, quad, 0)
    plsc.subcore_barrier()
    pltpu.sync_copy(acc_sh.at[pl.ds(s * RPT, RPT)],
                    out_hbm.at[c].at[pl.ds(s * RPT, RPT)])


def _tc1_body(x_ref, w1_ref, wres_ref, bres_ref, degp_ref,
              hs_ref, xres_ref, dinv_ref):
    deg = jnp.sum(degp_ref[...], axis=1, keepdims=True) + 1.0
    dinv = lax.rsqrt(deg)
    x = x_ref[...]
    h = jnp.dot(x, w1_ref[...], preferred_element_type=f32)
    hs_ref[...] = h * dinv
    xres_ref[...] = jnp.dot(x, wres_ref[...], preferred_element_type=f32) + bres_ref[...]
    dinv_ref[...] = dinv


_tc1 = pl.pallas_call(
    _tc1_body,
    grid=(GRID,),
    in_specs=[
        pl.BlockSpec((R, F), lambda i: (i, 0)),
        pl.BlockSpec((F, F), lambda i: (0, 0)),
        pl.BlockSpec((F, F), lambda i: (0, 0)),
        pl.BlockSpec((1, F), lambda i: (0, 0)),
        pl.BlockSpec((R, NW), lambda i: (i, 0)),
    ],
    out_specs=[
        pl.BlockSpec((R, F), lambda i: (i, 0)),
        pl.BlockSpec((R, F), lambda i: (i, 0)),
        pl.BlockSpec((R, 1), lambda i: (i, 0)),
    ],
    out_shape=[
        jax.ShapeDtypeStruct((N, F), f32),
        jax.ShapeDtypeStruct((N, F), f32),
        jax.ShapeDtypeStruct((N, 1), f32),
    ],
)


def _tc_mid_body(aggp_ref, hs_ref, dinv_ref, b_ref, xres_ref, w_ref, out_ref):
    dinv = dinv_ref[...]
    o = dinv * (aggp_ref[0] + aggp_ref[1] + hs_ref[...]) + b_ref[...]
    if xres_ref is not None:
        o = o + xres_ref[...]
    a = jnp.maximum(o, 0.0)
    out_ref[...] = jnp.dot(a, w_ref[...], preferred_element_type=f32) * dinv


def _make_mid(with_xres):
    def body(*refs):
        if with_xres:
            aggp, hs, dinv, b, xres, w, out = refs
        else:
            aggp, hs, dinv, b, w, out = refs
            xres = None
        _tc_mid_body(aggp, hs, dinv, b, xres, w, out)

    in_specs = [
        pl.BlockSpec((NC, R, F), lambda i: (0, i, 0)),
        pl.BlockSpec((R, F), lambda i: (i, 0)),
        pl.BlockSpec((R, 1), lambda i: (i, 0)),
        pl.BlockSpec((1, F), lambda i: (0, 0)),
    ]
    if with_xres:
        in_specs.append(pl.BlockSpec((R, F), lambda i: (i, 0)))
    in_specs.append(pl.BlockSpec((F, F), lambda i: (0, 0)))
    return pl.pallas_call(
        body,
        grid=(GRID,),
        in_specs=in_specs,
        out_specs=pl.BlockSpec((R, F), lambda i: (i, 0)),
        out_shape=jax.ShapeDtypeStruct((N, F), f32),
    )


_tc_mid_res = _make_mid(True)
_tc_mid = _make_mid(False)


def _tc_fin_body(aggp_ref, hs_ref, dinv_ref, b_ref, wlin_ref, blin_ref, out_ref):
    o = dinv_ref[...] * (aggp_ref[0] + aggp_ref[1] + hs_ref[...]) + b_ref[...]
    a = jnp.maximum(o, 0.0)
    out_ref[...] = jnp.dot(a, wlin_ref[...], preferred_element_type=f32) + blin_ref[...]


_tc_fin = pl.pallas_call(
    _tc_fin_body,
    grid=(GRID,),
    in_specs=[
        pl.BlockSpec((NC, R, F), lambda i: (0, i, 0)),
        pl.BlockSpec((R, F), lambda i: (i, 0)),
        pl.BlockSpec((R, 1), lambda i: (i, 0)),
        pl.BlockSpec((1, F), lambda i: (0, 0)),
        pl.BlockSpec((F, F), lambda i: (0, 0)),
        pl.BlockSpec((1, F), lambda i: (0, 0)),
    ],
    out_specs=pl.BlockSpec((R, F), lambda i: (i, 0)),
    out_shape=jax.ShapeDtypeStruct((N, F), f32),
)


def kernel(x, edge_index, W1, b1, W2, b2, W3, b3, Wres, bres, Wlin, blin):
    src = edge_index[0]
    dst = edge_index[1]
    src_r = src.reshape(NW, S, K)
    dst_r = dst.reshape(NW, S, K)
    zrows = jnp.zeros((RPT, F), f32)

    deg_k = _get_deg_kernel()
    agg_k = _get_agg_kernel()
    degp = deg_k(dst)
    hs1, xres, dinv = _tc1(x, W1, Wres, bres.reshape(1, F), degp.T)
    agg1 = agg_k(hs1, src_r, dst_r, zrows)
    hs2 = _tc_mid_res(agg1, hs1, dinv, b1.reshape(1, F), xres, W2)
    agg2 = agg_k(hs2, src_r, dst_r, zrows)
    hs3 = _tc_mid(agg2, hs2, dinv, b2.reshape(1, F), W3)
    agg3 = agg_k(hs3, src_r, dst_r, zrows)
    out = _tc_fin(agg3, hs3, dinv, b3.reshape(1, F), Wlin, blin.reshape(1, F))
    return out


# speedup vs baseline: 1.1082x; 1.0023x over previous
import functools

import jax
import jax.numpy as jnp
from jax import lax
from jax.experimental import pallas as pl
from jax.experimental.pallas import tpu as pltpu
from jax.experimental.pallas import tpu_sc as plsc

N = 10000
E = 320000
F = 128

NC = 2
NS = 16
NW = NC * NS
EPW = E // NW
K = 50
S = EPW // K
RPT = N // NS

R = 5000
GRID = N // R

f32 = jnp.float32
i32 = jnp.int32


_SC_PARAMS = pltpu.CompilerParams(needs_layout_passes=False,
                                  use_tc_tiling_on_sc=False)


@functools.cache
def _sc_mesh():
    return plsc.VectorSubcoreMesh(core_axis_name="c", subcore_axis_name="s",
                                  num_cores=NC, num_subcores=NS)


@functools.cache
def _get_deg_kernel():
    return pl.kernel(
        _deg_body,
        out_type=jax.ShapeDtypeStruct((NW, N), f32),
        mesh=_sc_mesh(),
        scratch_types=[
            pltpu.VMEM((EPW,), i32),
            pltpu.VMEM((N,), f32),
        ],
        compiler_params=_SC_PARAMS,
    )


def _deg_body(dst_hbm, out_hbm, dst_v, deg_v):
    w = lax.axis_index("c") * NS + lax.axis_index("s")
    pltpu.sync_copy(dst_hbm.at[pl.ds(w * EPW, EPW)], dst_v)

    zeros = jnp.zeros((16,), f32)

    def zero_body(i, carry):
        deg_v[pl.ds(i * 16, 16)] = zeros
        return carry

    lax.fori_loop(0, N // 16, zero_body, 0)

    ones = jnp.ones((16,), f32)

    def body(i, carry):
        idx = dst_v[pl.ds(i * 16, 16)]
        plsc.addupdate_scatter(deg_v, [idx], ones)
        return carry

    lax.fori_loop(0, EPW // 16, body, 0)
    pltpu.sync_copy(deg_v, out_hbm.at[w])


@functools.cache
def _get_agg_kernel():
    return pl.kernel(
        _agg_body,
        out_type=jax.ShapeDtypeStruct((NC, N, F), f32),
        mesh=_sc_mesh(),
        scratch_types=[
            pltpu.VMEM((S, K), i32),
            pltpu.VMEM((S, K), i32),
            pltpu.VMEM((K, F), f32),
            pltpu.VMEM((K, F), f32),
            pltpu.VMEM((K, F), f32),
            pltpu.VMEM((K, F), f32),
            pltpu.VMEM_SHARED((N, F), f32),
            pltpu.SemaphoreType.DMA,
            pltpu.SemaphoreType.DMA,
            pltpu.SemaphoreType.DMA,
            pltpu.SemaphoreType.DMA,
            pltpu.SemaphoreType.DMA,
            pltpu.SemaphoreType.DMA,
            pltpu.SemaphoreType.DMA,
            pltpu.SemaphoreType.DMA,
        ],
        compiler_params=_SC_PARAMS,
    )


def _agg_body(hs_hbm, srcr_hbm, dstr_hbm, zrows_hbm, out_hbm,
              src_v, dst_v, buf0, buf1, buf2, buf3, acc_sh,
              gs0, gs1, gs2, gs3, ss0, ss1, ss2, ss3):
    bufs = (buf0, buf1, buf2, buf3)
    gsems = (gs0, gs1, gs2, gs3)
    ssems = (ss0, ss1, ss2, ss3)
    c = lax.axis_index("c")
    s = lax.axis_index("s")
    w = c * NS + s
    pltpu.sync_copy(srcr_hbm.at[w], src_v)
    pltpu.sync_copy(dstr_hbm.at[w], dst_v)
    pltpu.sync_copy(zrows_hbm, acc_sh.at[pl.ds(s * RPT, RPT)])
    plsc.subcore_barrier()

    D = 4
    for b in range(D):
        pltpu.async_copy(hs_hbm.at[src_v.at[b]], bufs[b], gsems[b])

    def quad(q, carry):
        for b in range(D):
            g = D * q + b
            pltpu.make_async_copy(hs_hbm.at[src_v.at[g]], bufs[b], gsems[b]).wait()
            sc = pltpu.async_copy(bufs[b], acc_sh.at[dst_v.at[g]], ssems[b],
                                  add=True)
            sc.wait()

            @pl.when(g + D < S)
            def _():
                pltpu.async_copy(hs_hbm.at[src_v.at[g + D]], bufs[b], gsems[b])
        return carry

    lax.fori_loop(0, S // D, quad, 0)
    plsc.subcore_barrier()
    pltpu.sync_copy(acc_sh.at[pl.ds(s * RPT, RPT)],
                    out_hbm.at[c].at[pl.ds(s * RPT, RPT)])


def _tc1_body(x_ref, w1_ref, wres_ref, bres_ref, degp_ref,
              hs_ref, xres_ref, dinv_ref):
    deg = jnp.sum(degp_ref[...], axis=1, keepdims=True) + 1.0
    dinv = lax.rsqrt(deg)
    x = x_ref[...]
    h = jnp.dot(x, w1_ref[...], preferred_element_type=f32)
    hs_ref[...] = h * dinv
    xres_ref[...] = jnp.dot(x, wres_ref[...], preferred_element_type=f32) + bres_ref[...]
    dinv_ref[...] = dinv


_tc1 = pl.pallas_call(
    _tc1_body,
    grid=(GRID,),
    in_specs=[
        pl.BlockSpec((R, F), lambda i: (i, 0)),
        pl.BlockSpec((F, F), lambda i: (0, 0)),
        pl.BlockSpec((F, F), lambda i: (0, 0)),
        pl.BlockSpec((1, F), lambda i: (0, 0)),
        pl.BlockSpec((R, NW), lambda i: (i, 0)),
    ],
    out_specs=[
        pl.BlockSpec((R, F), lambda i: (i, 0)),
        pl.BlockSpec((R, F), lambda i: (i, 0)),
        pl.BlockSpec((R, 1), lambda i: (i, 0)),
    ],
    out_shape=[
        jax.ShapeDtypeStruct((N, F), f32),
        jax.ShapeDtypeStruct((N, F), f32),
        jax.ShapeDtypeStruct((N, 1), f32),
    ],
)


def _tc_mid_body(aggp_ref, hs_ref, dinv_ref, b_ref, xres_ref, w_ref, out_ref):
    dinv = dinv_ref[...]
    o = dinv * (aggp_ref[0] + aggp_ref[1] + hs_ref[...]) + b_ref[...]
    if xres_ref is not None:
        o = o + xres_ref[...]
    a = jnp.maximum(o, 0.0)
    out_ref[...] = jnp.dot(a, w_ref[...], preferred_element_type=f32) * dinv


def _make_mid(with_xres):
    def body(*refs):
        if with_xres:
            aggp, hs, dinv, b, xres, w, out = refs
        else:
            aggp, hs, dinv, b, w, out = refs
            xres = None
        _tc_mid_body(aggp, hs, dinv, b, xres, w, out)

    in_specs = [
        pl.BlockSpec((NC, R, F), lambda i: (0, i, 0)),
        pl.BlockSpec((R, F), lambda i: (i, 0)),
        pl.BlockSpec((R, 1), lambda i: (i, 0)),
        pl.BlockSpec((1, F), lambda i: (0, 0)),
    ]
    if with_xres:
        in_specs.append(pl.BlockSpec((R, F), lambda i: (i, 0)))
    in_specs.append(pl.BlockSpec((F, F), lambda i: (0, 0)))
    return pl.pallas_call(
        body,
        grid=(GRID,),
        in_specs=in_specs,
        out_specs=pl.BlockSpec((R, F), lambda i: (i, 0)),
        out_shape=jax.ShapeDtypeStruct((N, F), f32),
    )


_tc_mid_res = _make_mid(True)
_tc_mid = _make_mid(False)


def _tc_fin_body(aggp_ref, hs_ref, dinv_ref, b_ref, wlin_ref, blin_ref, out_ref):
    o = dinv_ref[...] * (aggp_ref[0] + aggp_ref[1] + hs_ref[...]) + b_ref[...]
    a = jnp.maximum(o, 0.0)
    out_ref[...] = jnp.dot(a, wlin_ref[...], preferred_element_type=f32) + blin_ref[...]


_tc_fin = pl.pallas_call(
    _tc_fin_body,
    grid=(GRID,),
    in_specs=[
        pl.BlockSpec((NC, R, F), lambda i: (0, i, 0)),
        pl.BlockSpec((R, F), lambda i: (i, 0)),
        pl.BlockSpec((R, 1), lambda i: (i, 0)),
        pl.BlockSpec((1, F), lambda i: (0, 0)),
        pl.BlockSpec((F, F), lambda i: (0, 0)),
        pl.BlockSpec((1, F), lambda i: (0, 0)),
    ],
    out_specs=pl.BlockSpec((R, F), lambda i: (i, 0)),
    out_shape=jax.ShapeDtypeStruct((N, F), f32),
)


def kernel(x, edge_index, W1, b1, W2, b2, W3, b3, Wres, bres, Wlin, blin):
    src = edge_index[0]
    dst = edge_index[1]
    src_r = src.reshape(NW, S, K)
    dst_r = dst.reshape(NW, S, K)
    zrows = jnp.zeros((RPT, F), f32)

    deg_k = _get_deg_kernel()
    agg_k = _get_agg_kernel()
    degp = deg_k(dst)
    hs1, xres, dinv = _tc1(x, W1, Wres, bres.reshape(1, F), degp.T)
    agg1 = agg_k(hs1, src_r, dst_r, zrows)
    hs2 = _tc_mid_res(agg1, hs1, dinv, b1.reshape(1, F), xres, W2)
    agg2 = agg_k(hs2, src_r, dst_r, zrows)
    hs3 = _tc_mid(agg2, hs2, dinv, b2.reshape(1, F), W3)
    agg3 = agg_k(hs3, src_r, dst_r, zrows)
    out = _tc_fin(agg3, hs3, dinv, b3.reshape(1, F), Wlin, blin.reshape(1, F))
    return out


# speedup vs baseline: 1.1279x; 1.0177x over previous
import functools

import jax
import jax.numpy as jnp
from jax import lax
from jax.experimental import pallas as pl
from jax.experimental.pallas import tpu as pltpu
from jax.experimental.pallas import tpu_sc as plsc

N = 10000
E = 320000
F = 128

NC = 2
NS = 16
NW = NC * NS
EPW = E // NW
K = 50
S = EPW // K
RPT = N // NS

R = 5000
GRID = N // R

f32 = jnp.float32
i32 = jnp.int32


_SC_PARAMS = pltpu.CompilerParams(needs_layout_passes=False,
                                  use_tc_tiling_on_sc=False)


@functools.cache
def _sc_mesh():
    return plsc.VectorSubcoreMesh(core_axis_name="c", subcore_axis_name="s",
                                  num_cores=NC, num_subcores=NS)


@functools.cache
def _get_deg_kernel():
    return pl.kernel(
        _deg_body,
        out_type=jax.ShapeDtypeStruct((NW, N), f32),
        mesh=_sc_mesh(),
        scratch_types=[
            pltpu.VMEM((EPW,), i32),
            pltpu.VMEM((N,), f32),
        ],
        compiler_params=_SC_PARAMS,
    )


def _deg_body(dst_hbm, zflat_hbm, out_hbm, dst_v, deg_v):
    w = lax.axis_index("c") * NS + lax.axis_index("s")
    pltpu.sync_copy(dst_hbm.at[pl.ds(w * EPW, EPW)], dst_v)
    pltpu.sync_copy(zflat_hbm, deg_v)

    ones = jnp.ones((16,), f32)

    def body(i, carry):
        idx = dst_v[pl.ds(i * 16, 16)]
        plsc.addupdate_scatter(deg_v, [idx], ones)
        return carry

    lax.fori_loop(0, EPW // 16, body, 0)
    pltpu.sync_copy(deg_v, out_hbm.at[w])


@functools.cache
def _get_agg_kernel():
    return pl.kernel(
        _agg_body,
        out_type=jax.ShapeDtypeStruct((NC, N, F), f32),
        mesh=_sc_mesh(),
        scratch_types=[
            pltpu.VMEM((S, K), i32),
            pltpu.VMEM((S, K), i32),
            pltpu.VMEM((K, F), f32),
            pltpu.VMEM((K, F), f32),
            pltpu.VMEM((K, F), f32),
            pltpu.VMEM((K, F), f32),
            pltpu.VMEM_SHARED((N, F), f32),
            pltpu.SemaphoreType.DMA,
            pltpu.SemaphoreType.DMA,
            pltpu.SemaphoreType.DMA,
            pltpu.SemaphoreType.DMA,
            pltpu.SemaphoreType.DMA,
            pltpu.SemaphoreType.DMA,
            pltpu.SemaphoreType.DMA,
            pltpu.SemaphoreType.DMA,
        ],
        compiler_params=_SC_PARAMS,
    )


def _agg_body(hs_hbm, srcr_hbm, dstr_hbm, out_hbm,
              src_v, dst_v, buf0, buf1, buf2, buf3, acc_sh,
              gs0, gs1, gs2, gs3, ss0, ss1, ss2, ss3):
    bufs = (buf0, buf1, buf2, buf3)
    gsems = (gs0, gs1, gs2, gs3)
    ssems = (ss0, ss1, ss2, ss3)
    c = lax.axis_index("c")
    s = lax.axis_index("s")
    w = c * NS + s
    pltpu.sync_copy(srcr_hbm.at[w], src_v)
    pltpu.sync_copy(dstr_hbm.at[w], dst_v)

    zeros = jnp.zeros((16,), f32)

    def zrow(i, carry):
        buf0[i // 8, pl.ds((i % 8) * 16, 16)] = zeros
        return carry

    lax.fori_loop(0, K * 8, zrow, 0)
    for j in range(RPT // K):
        pltpu.sync_copy(buf0, acc_sh.at[pl.ds(s * RPT + j * K, K)])
    rem = RPT % K
    if rem:
        pltpu.sync_copy(buf0.at[pl.ds(0, rem)],
                        acc_sh.at[pl.ds(s * RPT + (RPT // K) * K, rem)])
    plsc.subcore_barrier()

    D = 4
    for b in range(D):
        pltpu.async_copy(hs_hbm.at[src_v.at[b]], bufs[b], gsems[b])

    def quad(q, carry):
        for b in range(D):
            g = D * q + b
            pltpu.make_async_copy(hs_hbm.at[src_v.at[g]], bufs[b], gsems[b]).wait()
            sc = pltpu.async_copy(bufs[b], acc_sh.at[dst_v.at[g]], ssems[b],
                                  add=True)
            sc.wait()

            @pl.when(g + D < S)
            def _():
                pltpu.async_copy(hs_hbm.at[src_v.at[g + D]], bufs[b], gsems[b])
        return carry

    lax.fori_loop(0, S // D, quad, 0)
    plsc.subcore_barrier()
    pltpu.sync_copy(acc_sh.at[pl.ds(s * RPT, RPT)],
                    out_hbm.at[c].at[pl.ds(s * RPT, RPT)])


def _tc1_body(x_ref, w1_ref, wres_ref, bres_ref, degp_ref,
              hs_ref, xres_ref, dinv_ref):
    deg = jnp.sum(degp_ref[...], axis=1, keepdims=True) + 1.0
    dinv = lax.rsqrt(deg)
    x = x_ref[...]
    h = jnp.dot(x, w1_ref[...], preferred_element_type=f32)
    hs_ref[...] = h * dinv
    xres_ref[...] = jnp.dot(x, wres_ref[...], preferred_element_type=f32) + bres_ref[...]
    dinv_ref[...] = dinv


_tc1 = pl.pallas_call(
    _tc1_body,
    grid=(GRID,),
    in_specs=[
        pl.BlockSpec((R, F), lambda i: (i, 0)),
        pl.BlockSpec((F, F), lambda i: (0, 0)),
        pl.BlockSpec((F, F), lambda i: (0, 0)),
        pl.BlockSpec((1, F), lambda i: (0, 0)),
        pl.BlockSpec((R, NW), lambda i: (i, 0)),
    ],
    out_specs=[
        pl.BlockSpec((R, F), lambda i: (i, 0)),
        pl.BlockSpec((R, F), lambda i: (i, 0)),
        pl.BlockSpec((R, 1), lambda i: (i, 0)),
    ],
    out_shape=[
        jax.ShapeDtypeStruct((N, F), f32),
        jax.ShapeDtypeStruct((N, F), f32),
        jax.ShapeDtypeStruct((N, 1), f32),
    ],
)


def _tc_mid_body(aggp_ref, hs_ref, dinv_ref, b_ref, xres_ref, w_ref, out_ref):
    dinv = dinv_ref[...]
    o = dinv * (aggp_ref[0] + aggp_ref[1] + hs_ref[...]) + b_ref[...]
    if xres_ref is not None:
        o = o + xres_ref[...]
    a = jnp.maximum(o, 0.0)
    out_ref[...] = jnp.dot(a, w_ref[...], preferred_element_type=f32) * dinv


def _make_mid(with_xres):
    def body(*refs):
        if with_xres:
            aggp, hs, dinv, b, xres, w, out = refs
        else:
            aggp, hs, dinv, b, w, out = refs
            xres = None
        _tc_mid_body(aggp, hs, dinv, b, xres, w, out)

    in_specs = [
        pl.BlockSpec((NC, R, F), lambda i: (0, i, 0)),
        pl.BlockSpec((R, F), lambda i: (i, 0)),
        pl.BlockSpec((R, 1), lambda i: (i, 0)),
        pl.BlockSpec((1, F), lambda i: (0, 0)),
    ]
    if with_xres:
        in_specs.append(pl.BlockSpec((R, F), lambda i: (i, 0)))
    in_specs.append(pl.BlockSpec((F, F), lambda i: (0, 0)))
    return pl.pallas_call(
        body,
        grid=(GRID,),
        in_specs=in_specs,
        out_specs=pl.BlockSpec((R, F), lambda i: (i, 0)),
        out_shape=jax.ShapeDtypeStruct((N, F), f32),
    )


_tc_mid_res = _make_mid(True)
_tc_mid = _make_mid(False)


def _tc_fin_body(aggp_ref, hs_ref, dinv_ref, b_ref, wlin_ref, blin_ref, out_ref):
    o = dinv_ref[...] * (aggp_ref[0] + aggp_ref[1] + hs_ref[...]) + b_ref[...]
    a = jnp.maximum(o, 0.0)
    out_ref[...] = jnp.dot(a, wlin_ref[...], preferred_element_type=f32) + blin_ref[...]


_tc_fin = pl.pallas_call(
    _tc_fin_body,
    grid=(GRID,),
    in_specs=[
        pl.BlockSpec((NC, R, F), lambda i: (0, i, 0)),
        pl.BlockSpec((R, F), lambda i: (i, 0)),
        pl.BlockSpec((R, 1), lambda i: (i, 0)),
        pl.BlockSpec((1, F), lambda i: (0, 0)),
        pl.BlockSpec((F, F), lambda i: (0, 0)),
        pl.BlockSpec((1, F), lambda i: (0, 0)),
    ],
    out_specs=pl.BlockSpec((R, F), lambda i: (i, 0)),
    out_shape=jax.ShapeDtypeStruct((N, F), f32),
)


def kernel(x, edge_index, W1, b1, W2, b2, W3, b3, Wres, bres, Wlin, blin):
    src = edge_index[0]
    dst = edge_index[1]
    src_r = src.reshape(NW, S, K)
    dst_r = dst.reshape(NW, S, K)
    zflat = jnp.zeros((N,), f32)

    deg_k = _get_deg_kernel()
    agg_k = _get_agg_kernel()
    degp = deg_k(dst, zflat)
    hs1, xres, dinv = _tc1(x, W1, Wres, bres.reshape(1, F), degp.T)
    agg1 = agg_k(hs1, src_r, dst_r)
    hs2 = _tc_mid_res(agg1, hs1, dinv, b1.reshape(1, F), xres, W2)
    agg2 = agg_k(hs2, src_r, dst_r)
    hs3 = _tc_mid(agg2, hs2, dinv, b2.reshape(1, F), W3)
    agg3 = agg_k(hs3, src_r, dst_r)
    out = _tc_fin(agg3, hs3, dinv, b3.reshape(1, F), Wlin, blin.reshape(1, F))
    return out
